# baseline (speedup 1.0000x reference)
import jax
import jax.numpy as jnp
from jax import lax
from jax.experimental import pallas as pl
from jax.experimental.pallas import tpu as pltpu
from jax.experimental.pallas import tpu_sc as plsc


def _gather_codes(codes_p, idx):
    n = idx.shape[0]
    _, w = codes_p.shape
    info = plsc.get_sparse_core_info()
    nc, ns = info.num_cores, info.num_subcores
    nw = nc * ns
    n_per_w = n // nw
    ch = 800
    nch = n_per_w // ch

    mesh = plsc.VectorSubcoreMesh(core_axis_name="c", subcore_axis_name="s")

    def body(codes_hbm, idx_hbm, out_hbm, idx_v, rows_v, sem):
        wid = lax.axis_index("s") * nc + lax.axis_index("c")
        base = wid * n_per_w

        def step(i, carry):
            off = base + i * ch
            pltpu.sync_copy(idx_hbm.at[pl.ds(off, ch)], idx_v)
            pltpu.async_copy(codes_hbm.at[idx_v], rows_v, sem).wait()
            pltpu.sync_copy(rows_v, out_hbm.at[pl.ds(off, ch)])
            return carry

        lax.fori_loop(0, nch, step, 0)

    f = pl.kernel(
        body,
        mesh=mesh,
        out_type=jax.ShapeDtypeStruct((n, w), jnp.int32),
        scratch_types=[
            pltpu.VMEM((ch,), jnp.int32),
            pltpu.VMEM((ch, w), jnp.int32),
            pltpu.SemaphoreType.DMA,
        ],
        compiler_params=pltpu.CompilerParams(use_tc_tiling_on_sc=True),
    )
    return f(codes_p, idx)


def _combine(wc, cbt, m, t=1600, interpret=False):
    n, w = wc.shape
    d, mk = cbt.shape
    k = mk // m
    grid = n // t

    def body(wc_ref, cbt_ref, out_ref):
        one = jnp.bfloat16(1.0)
        zero = jnp.bfloat16(0.0)
        wcs = wc_ref[...][:, :m].T.astype(jnp.int16)
        iota = lax.broadcasted_iota(jnp.int16, (k, t), 0)

        def onehot(j):
            row = lax.broadcast_in_dim(wcs[j : j + 1, :], (k, t), (0, 1))
            return jnp.where(row == iota, one, zero)

        acc = jnp.zeros((d, t), jnp.float32)
        for j in range(m):
            acc = acc + lax.dot_general(
                cbt_ref[:, j * k : (j + 1) * k], onehot(j),
                (((1,), (0,)), ((), ())),
                preferred_element_type=jnp.float32)
        out_ref[...] = acc.T

    return pl.pallas_call(
        body,
        grid=(grid,),
        in_specs=[
            pl.BlockSpec((t, w), lambda i: (i, 0)),
            pl.BlockSpec((d, mk), lambda i: (0, 0)),
        ],
        out_specs=pl.BlockSpec((t, d), lambda i: (i, 0)),
        out_shape=jax.ShapeDtypeStruct((n, d), jnp.float32),
        compiler_params=pltpu.CompilerParams(
            dimension_semantics=("arbitrary",)),
        interpret=interpret,
    )(wc, cbt)


def kernel(x, codes, codebook):
    b, l = x.shape
    m, k, d = codebook.shape
    n = b * l
    codes_p = jnp.pad(codes, ((0, 0), (0, 128 - codes.shape[1])))
    wc = _gather_codes(codes_p, x.reshape(n))
    cbt = codebook.transpose(2, 0, 1).reshape(d, m * k).astype(jnp.bfloat16)
    out = _combine(wc, cbt, m)
    return out.reshape(b, l, d)


# speedup vs baseline: 1.0814x; 1.0814x over previous
import jax
import jax.numpy as jnp
from jax import lax
from jax.experimental import pallas as pl
from jax.experimental.pallas import tpu as pltpu
from jax.experimental.pallas import tpu_sc as plsc


def _gather_codes(codes_p, idx):
    n = idx.shape[0]
    _, w = codes_p.shape
    info = plsc.get_sparse_core_info()
    nc, ns = info.num_cores, info.num_subcores
    nw = nc * ns
    n_per_w = n // nw
    ch = 800
    nch = n_per_w // ch

    mesh = plsc.VectorSubcoreMesh(core_axis_name="c", subcore_axis_name="s")

    def body(codes_hbm, idx_hbm, out_hbm, idx_v, rows_v, sem):
        wid = lax.axis_index("s") * nc + lax.axis_index("c")
        base = wid * n_per_w

        def step(i, carry):
            off = base + i * ch
            pltpu.sync_copy(idx_hbm.at[pl.ds(off, ch)], idx_v)
            pltpu.async_copy(codes_hbm.at[idx_v], rows_v, sem).wait()
            pltpu.sync_copy(rows_v, out_hbm.at[pl.ds(off, ch)])
            return carry

        lax.fori_loop(0, nch, step, 0)

    f = pl.kernel(
        body,
        mesh=mesh,
        out_type=jax.ShapeDtypeStruct((n, w), jnp.int32),
        scratch_types=[
            pltpu.VMEM((ch,), jnp.int32),
            pltpu.VMEM((ch, w), jnp.int32),
            pltpu.SemaphoreType.DMA,
        ],
        compiler_params=pltpu.CompilerParams(use_tc_tiling_on_sc=True),
    )
    return f(codes_p, idx)


def _combine(wc, cbt, m, t=2048, interpret=False):
    n, w = wc.shape
    d, mk = cbt.shape
    k = mk // m
    grid = n // t

    def body(wc_ref, cbt_ref, out_ref):
        one = jnp.bfloat16(1.0)
        zero = jnp.bfloat16(0.0)
        wcs = wc_ref[...][:, :m].T.astype(jnp.int16)
        iota = lax.broadcasted_iota(jnp.int16, (k, t), 0)

        def onehot(j):
            row = lax.broadcast_in_dim(wcs[j : j + 1, :], (k, t), (0, 1))
            return jnp.where(row == iota, one, zero)

        acc = jnp.zeros((d, t), jnp.float32)
        for j in range(m):
            acc = acc + lax.dot_general(
                cbt_ref[:, j * k : (j + 1) * k], onehot(j),
                (((1,), (0,)), ((), ())),
                preferred_element_type=jnp.float32)
        out_ref[...] = acc.T

    return pl.pallas_call(
        body,
        grid=(grid,),
        in_specs=[
            pl.BlockSpec((t, w), lambda i: (i, 0)),
            pl.BlockSpec((d, mk), lambda i: (0, 0)),
        ],
        out_specs=pl.BlockSpec((t, d), lambda i: (i, 0)),
        out_shape=jax.ShapeDtypeStruct((n, d), jnp.float32),
        compiler_params=pltpu.CompilerParams(
            dimension_semantics=("arbitrary",)),
        interpret=interpret,
    )(wc, cbt)


def kernel(x, codes, codebook):
    b, l = x.shape
    m, k, d = codebook.shape
    n = b * l
    codes_p = jnp.pad(codes, ((0, 0), (0, 128 - codes.shape[1])))
    wc = _gather_codes(codes_p, x.reshape(n))
    cbt = codebook.transpose(2, 0, 1).reshape(d, m * k).astype(jnp.bfloat16)
    out = _combine(wc, cbt, m)
    return out.reshape(b, l, d)


# speedup vs baseline: 1.2203x; 1.1284x over previous
import jax
import jax.numpy as jnp
from jax import lax
from jax.experimental import pallas as pl
from jax.experimental.pallas import tpu as pltpu
from jax.experimental.pallas import tpu_sc as plsc


def _combine(wc, cbt, m, t=2048, interpret=False):
    n, w = wc.shape
    d, mk = cbt.shape
    k = mk // m
    grid = n // t

    def body(wc_ref, cbt_ref, out_ref):
        one = jnp.bfloat16(1.0)
        zero = jnp.bfloat16(0.0)
        wcs = wc_ref[...][:, :m].T.astype(jnp.int16)
        iota = lax.broadcasted_iota(jnp.int16, (k, t), 0)

        def onehot(j):
            row = lax.broadcast_in_dim(wcs[j : j + 1, :], (k, t), (0, 1))
            return jnp.where(row == iota, one, zero)

        acc = jnp.zeros((d, t), jnp.float32)
        for j in range(m):
            acc = acc + lax.dot_general(
                cbt_ref[:, j * k : (j + 1) * k], onehot(j),
                (((1,), (0,)), ((), ())),
                preferred_element_type=jnp.float32)
        out_ref[...] = acc.T

    return pl.pallas_call(
        body,
        grid=(grid,),
        in_specs=[
            pl.BlockSpec((t, w), lambda i: (i, 0)),
            pl.BlockSpec((d, mk), lambda i: (0, 0)),
        ],
        out_specs=pl.BlockSpec((t, d), lambda i: (i, 0)),
        out_shape=jax.ShapeDtypeStruct((n, d), jnp.float32),
        compiler_params=pltpu.CompilerParams(
            dimension_semantics=("arbitrary",)),
        interpret=interpret,
    )(wc, cbt)


def _gather_rows(tbl, idx):
    n = idx.shape[0]
    _, w = tbl.shape
    info = plsc.get_sparse_core_info()
    nc, ns = info.num_cores, info.num_subcores
    nw = nc * ns
    n_per_w = n // nw
    ch = 800
    nch = n_per_w // ch

    mesh = plsc.VectorSubcoreMesh(core_axis_name="c", subcore_axis_name="s")

    def body(tbl_hbm, idx_hbm, out_hbm, idx_v, rows_v, sem):
        wid = lax.axis_index("s") * nc + lax.axis_index("c")
        base = wid * n_per_w

        def step(i, carry):
            off = base + i * ch
            pltpu.sync_copy(idx_hbm.at[pl.ds(off, ch)], idx_v)
            pltpu.async_copy(tbl_hbm.at[idx_v], rows_v, sem).wait()
            pltpu.sync_copy(rows_v, out_hbm.at[pl.ds(off, ch)])
            return carry

        lax.fori_loop(0, nch, step, 0)

    f = pl.kernel(
        body,
        mesh=mesh,
        out_type=jax.ShapeDtypeStruct((n, w), jnp.float32),
        scratch_types=[
            pltpu.VMEM((ch,), jnp.int32),
            pltpu.VMEM((ch, w), jnp.float32),
            pltpu.SemaphoreType.DMA,
        ],
        compiler_params=pltpu.CompilerParams(use_tc_tiling_on_sc=True),
    )
    return f(tbl, idx)


def kernel(x, codes, codebook):
    b, l = x.shape
    m, k, d = codebook.shape
    v = codes.shape[0]
    n = b * l
    t = 2048
    vp = ((v + t - 1) // t) * t
    codes_p = jnp.pad(codes, ((0, vp - v), (0, 128 - codes.shape[1])))
    cbt = codebook.transpose(2, 0, 1).reshape(d, m * k).astype(jnp.bfloat16)
    emb = _combine(codes_p, cbt, m, t=t)
    emb2 = emb.reshape(vp * 2, d // 2)
    xi = x.reshape(n)
    idx2 = jnp.stack([2 * xi, 2 * xi + 1], axis=1).reshape(2 * n)
    out2 = _gather_rows(emb2, idx2)
    return out2.reshape(b, l, d)


# speedup vs baseline: 1.4931x; 1.2235x over previous
import jax
import jax.numpy as jnp
from jax import lax
from jax.experimental import pallas as pl
from jax.experimental.pallas import tpu as pltpu
from jax.experimental.pallas import tpu_sc as plsc


def _combine(wc, cbt, m, t=2048, interpret=False):
    n, w = wc.shape
    d, mk = cbt.shape
    k = mk // m
    grid = n // t

    def body(wc_ref, cbt_ref, out_ref):
        one = jnp.bfloat16(1.0)
        zero = jnp.bfloat16(0.0)
        wcs = wc_ref[...][:, :m].T.astype(jnp.int16)
        iota = lax.broadcasted_iota(jnp.int16, (k, t), 0)

        def onehot(j):
            row = lax.broadcast_in_dim(wcs[j : j + 1, :], (k, t), (0, 1))
            return jnp.where(row == iota, one, zero)

        acc = jnp.zeros((d, t), jnp.float32)
        for j in range(m):
            acc = acc + lax.dot_general(
                cbt_ref[:, j * k : (j + 1) * k], onehot(j),
                (((1,), (0,)), ((), ())),
                preferred_element_type=jnp.float32)
        out_ref[...] = acc.T

    return pl.pallas_call(
        body,
        grid=(grid,),
        in_specs=[
            pl.BlockSpec((t, w), lambda i: (i, 0)),
            pl.BlockSpec((d, mk), lambda i: (0, 0)),
        ],
        out_specs=pl.BlockSpec((t, d), lambda i: (i, 0)),
        out_shape=jax.ShapeDtypeStruct((n, d), jnp.float32),
        compiler_params=pltpu.CompilerParams(
            dimension_semantics=("arbitrary",)),
        interpret=interpret,
    )(wc, cbt)


def _gather_rows(tbl, idx):
    n = idx.shape[0]
    _, w = tbl.shape
    info = plsc.get_sparse_core_info()
    nc, ns = info.num_cores, info.num_subcores
    nw = nc * ns
    n_per_w = n // nw
    ch = 400
    nch = n_per_w // ch

    mesh = plsc.VectorSubcoreMesh(core_axis_name="c", subcore_axis_name="s")

    def body(tbl_hbm, idx_hbm, out_hbm, idx_v, rows_v, sem):
        wid = lax.axis_index("s") * nc + lax.axis_index("c")
        base = wid * n_per_w

        def step(i, carry):
            off = base + i * ch
            pltpu.sync_copy(idx_hbm.at[pl.ds(off, ch)], idx_v)
            pltpu.async_copy(tbl_hbm.at[idx_v], rows_v, sem).wait()
            pltpu.sync_copy(rows_v, out_hbm.at[pl.ds(off, ch)])
            return carry

        lax.fori_loop(0, nch, step, 0)

    f = pl.kernel(
        body,
        mesh=mesh,
        out_type=jax.ShapeDtypeStruct((n, w), jnp.float32),
        scratch_types=[
            pltpu.VMEM((ch,), jnp.int32),
            pltpu.VMEM((ch, w), jnp.float32),
            pltpu.SemaphoreType.DMA,
        ],
        compiler_params=pltpu.CompilerParams(use_tc_tiling_on_sc=True),
    )
    return f(tbl, idx)


def kernel(x, codes, codebook):
    b, l = x.shape
    m, k, d = codebook.shape
    v = codes.shape[0]
    n = b * l
    t = 2048
    vp = ((v + t - 1) // t) * t
    codes_p = jnp.pad(codes, ((0, vp - v), (0, 128 - codes.shape[1])))
    cbt = codebook.transpose(2, 0, 1).reshape(d, m * k).astype(jnp.bfloat16)
    emb = _combine(codes_p, cbt, m, t=t)
    out = _gather_rows(emb, x.reshape(n))
    return out.reshape(b, l, d)


# speedup vs baseline: 1.4977x; 1.0031x over previous
import jax
import jax.numpy as jnp
from jax import lax
from jax.experimental import pallas as pl
from jax.experimental.pallas import tpu as pltpu
from jax.experimental.pallas import tpu_sc as plsc


def _combine(wc, cbt, m, t=2048, interpret=False):
    n, w = wc.shape
    d, mk = cbt.shape
    k = mk // m
    grid = n // t

    def body(wc_ref, cbt_ref, out_ref):
        one = jnp.bfloat16(1.0)
        zero = jnp.bfloat16(0.0)
        wcs = wc_ref[...][:, :m].T.astype(jnp.int16)
        iota = lax.broadcasted_iota(jnp.int16, (k, t), 0)

        def onehot(j):
            row = lax.broadcast_in_dim(wcs[j : j + 1, :], (k, t), (0, 1))
            return jnp.where(row == iota, one, zero)

        acc = jnp.zeros((d, t), jnp.float32)
        for j in range(m):
            acc = acc + lax.dot_general(
                cbt_ref[:, j * k : (j + 1) * k], onehot(j),
                (((1,), (0,)), ((), ())),
                preferred_element_type=jnp.float32)
        out_ref[...] = acc.T

    return pl.pallas_call(
        body,
        grid=(grid,),
        in_specs=[
            pl.BlockSpec((t, w), lambda i: (i, 0)),
            pl.BlockSpec((d, mk), lambda i: (0, 0)),
        ],
        out_specs=pl.BlockSpec((t, d), lambda i: (i, 0)),
        out_shape=jax.ShapeDtypeStruct((n, d), jnp.float32),
        compiler_params=pltpu.CompilerParams(
            dimension_semantics=("arbitrary",)),
        interpret=interpret,
    )(wc, cbt)


def _gather_rows(tbl, idx):
    n = idx.shape[0]
    _, w = tbl.shape
    info = plsc.get_sparse_core_info()
    nc, ns = info.num_cores, info.num_subcores
    nw = nc * ns
    n_per_w = n // nw
    ch = 200
    nch = n_per_w // ch

    mesh = plsc.VectorSubcoreMesh(core_axis_name="c", subcore_axis_name="s")

    def body(tbl_hbm, idx_hbm, out_hbm, idx0, idx1, rows0, rows1, sem0, sem1):
        wid = lax.axis_index("s") * nc + lax.axis_index("c")
        base = wid * n_per_w
        idx_v = (idx0, idx1)
        rows_v = (rows0, rows1)
        sems = (sem0, sem1)

        pltpu.sync_copy(idx_hbm.at[pl.ds(base, ch)], idx_v[0])
        copies = {0: pltpu.async_copy(tbl_hbm.at[idx_v[0]], rows_v[0], sems[0])}
        for i in range(nch):
            cur = i % 2
            if i + 1 < nch:
                nxt = 1 - cur
                pltpu.sync_copy(
                    idx_hbm.at[pl.ds(base + (i + 1) * ch, ch)], idx_v[nxt])
                copies[i + 1] = pltpu.async_copy(
                    tbl_hbm.at[idx_v[nxt]], rows_v[nxt], sems[nxt])
            copies.pop(i).wait()
            pltpu.sync_copy(rows_v[cur], out_hbm.at[pl.ds(base + i * ch, ch)])

    f = pl.kernel(
        body,
        mesh=mesh,
        out_type=jax.ShapeDtypeStruct((n, w), jnp.float32),
        scratch_types=[
            pltpu.VMEM((ch,), jnp.int32),
            pltpu.VMEM((ch,), jnp.int32),
            pltpu.VMEM((ch, w), jnp.float32),
            pltpu.VMEM((ch, w), jnp.float32),
            pltpu.SemaphoreType.DMA,
            pltpu.SemaphoreType.DMA,
        ],
        compiler_params=pltpu.CompilerParams(use_tc_tiling_on_sc=True),
    )
    return f(tbl, idx)


def kernel(x, codes, codebook):
    b, l = x.shape
    m, k, d = codebook.shape
    v = codes.shape[0]
    n = b * l
    t = 2048
    vp = ((v + t - 1) // t) * t
    codes_p = jnp.pad(codes, ((0, vp - v), (0, 128 - codes.shape[1])))
    cbt = codebook.transpose(2, 0, 1).reshape(d, m * k).astype(jnp.bfloat16)
    emb = _combine(codes_p, cbt, m, t=t)
    out = _gather_rows(emb, x.reshape(n))
    return out.reshape(b, l, d)
